# baseline (speedup 1.0000x reference)
import functools

import jax
import jax.numpy as jnp
from jax import lax
from jax.experimental import pallas as pl
from jax.experimental.pallas import tpu as pltpu
from jax.experimental.pallas import tpu_sc as plsc

VOCAB = 1000000
DIM = 64
BSZ = 16384
SEQLEN = 200

NC = 2
NS = 16
NW = NC * NS

BTOT = BSZ * SEQLEN
PER_W = BTOT // NW
CHUNK = 512
NSTEP = PER_W // CHUNK


def _make_kernel():
    mesh = plsc.VectorSubcoreMesh(core_axis_name="c", subcore_axis_name="s")

    @functools.partial(
        pl.kernel,
        mesh=mesh,
        out_type=jax.ShapeDtypeStruct((BTOT, DIM), jnp.float32),
        scratch_types=[
            pltpu.VMEM((CHUNK,), jnp.int32),
            pltpu.VMEM((CHUNK, DIM), jnp.float32),
            pltpu.SemaphoreType.DMA,
        ],
        compiler_params=pltpu.CompilerParams(use_tc_tiling_on_sc=False),
    )
    def emb_kernel(idx_hbm, table_hbm, out_hbm, idx_v, rows_v, sem):
        wid = lax.axis_index("s") * NC + lax.axis_index("c")
        base = wid * PER_W

        def step(i, carry):
            off = base + i * CHUNK
            pltpu.sync_copy(idx_hbm.at[pl.ds(off, CHUNK)], idx_v)
            pltpu.async_copy(table_hbm.at[idx_v], rows_v, sem).wait()
            pltpu.sync_copy(rows_v, out_hbm.at[pl.ds(off, CHUNK)])
            return carry

        lax.fori_loop(0, NSTEP, step, 0)

    return emb_kernel


_EMB = _make_kernel()


def kernel(x, weight):
    flat = x.reshape(BTOT)
    out = _EMB(flat, weight)
    return out.reshape(BSZ, SEQLEN, DIM)


# speedup vs baseline: 1.0585x; 1.0585x over previous
import functools

import jax
import jax.numpy as jnp
from jax import lax
from jax.experimental import pallas as pl
from jax.experimental.pallas import tpu as pltpu
from jax.experimental.pallas import tpu_sc as plsc

VOCAB = 1000000
DIM = 64
BSZ = 16384
SEQLEN = 200

NC = 2
NS = 16
NW = NC * NS

BTOT = BSZ * SEQLEN
PER_W = BTOT // NW
CHUNK = 512
NSTEP = PER_W // CHUNK


def _make_kernel():
    mesh = plsc.VectorSubcoreMesh(core_axis_name="c", subcore_axis_name="s")

    @functools.partial(
        pl.kernel,
        mesh=mesh,
        out_type=jax.ShapeDtypeStruct((BTOT, DIM), jnp.float32),
        scratch_types=[
            pltpu.VMEM((CHUNK,), jnp.int32),
            pltpu.VMEM((CHUNK,), jnp.int32),
            pltpu.VMEM((CHUNK, DIM), jnp.float32),
            pltpu.VMEM((CHUNK, DIM), jnp.float32),
            pltpu.SemaphoreType.DMA,
            pltpu.SemaphoreType.DMA,
            pltpu.SemaphoreType.DMA,
            pltpu.SemaphoreType.DMA,
        ],
        compiler_params=pltpu.CompilerParams(use_tc_tiling_on_sc=False),
    )
    def emb_kernel(idx_hbm, table_hbm, out_hbm,
                   idx0, idx1, rows0, rows1, g0, g1, o0, o1):
        wid = lax.axis_index("s") * NC + lax.axis_index("c")
        base = wid * PER_W
        idx_b = (idx0, idx1)
        rows_b = (rows0, rows1)
        gsem = (g0, g1)
        osem = (o0, o1)

        def gstart(s, b):
            off = base + s * CHUNK
            pltpu.sync_copy(idx_hbm.at[pl.ds(off, CHUNK)], idx_b[b])
            pltpu.async_copy(table_hbm.at[idx_b[b]], rows_b[b], gsem[b])

        def gwait(b):
            pltpu.make_async_copy(
                table_hbm.at[idx_b[b]], rows_b[b], gsem[b]).wait()

        def ostart(s, b):
            off = base + s * CHUNK
            pltpu.async_copy(rows_b[b], out_hbm.at[pl.ds(off, CHUNK)], osem[b])

        def owait(s, b):
            off = base + s * CHUNK
            pltpu.make_async_copy(
                rows_b[b], out_hbm.at[pl.ds(off, CHUNK)], osem[b]).wait()

        gstart(0, 0)

        def body(j, carry):
            for b in range(2):
                s = 2 * j + b
                nb = 1 - b
                pl.when(s >= 1)(lambda: owait(s - 1, nb))
                pl.when(s + 1 < NSTEP)(lambda: gstart(s + 1, nb))
                gwait(b)
                ostart(s, b)
            return carry

        lax.fori_loop(0, NSTEP // 2, body, 0)
        owait(NSTEP - 1, 1)

    return emb_kernel


_EMB = _make_kernel()


def kernel(x, weight):
    flat = x.reshape(BTOT)
    out = _EMB(flat, weight)
    return out.reshape(BSZ, SEQLEN, DIM)


# speedup vs baseline: 1.0648x; 1.0060x over previous
import functools

import jax
import jax.numpy as jnp
from jax import lax
from jax.experimental import pallas as pl
from jax.experimental.pallas import tpu as pltpu
from jax.experimental.pallas import tpu_sc as plsc

VOCAB = 1000000
DIM = 64
BSZ = 16384
SEQLEN = 200

NC = 2
NS = 16
NW = NC * NS

ROWS_W = BSZ // NW
R = 4
NSTEP = ROWS_W // R


def _make_kernel():
    mesh = plsc.VectorSubcoreMesh(core_axis_name="c", subcore_axis_name="s")

    @functools.partial(
        pl.kernel,
        mesh=mesh,
        out_type=jax.ShapeDtypeStruct((BSZ, SEQLEN, DIM), jnp.float32),
        scratch_types=[
            pltpu.VMEM((R, SEQLEN), jnp.int32),
            pltpu.VMEM((R, SEQLEN), jnp.int32),
            pltpu.VMEM((R, SEQLEN, DIM), jnp.float32),
            pltpu.VMEM((R, SEQLEN, DIM), jnp.float32),
            pltpu.SemaphoreType.DMA,
            pltpu.SemaphoreType.DMA,
            pltpu.SemaphoreType.DMA,
            pltpu.SemaphoreType.DMA,
        ],
        compiler_params=pltpu.CompilerParams(use_tc_tiling_on_sc=False),
    )
    def emb_kernel(x_hbm, table_hbm, out_hbm,
                   idx0, idx1, rows0, rows1, g0, g1, o0, o1):
        wid = lax.axis_index("s") * NC + lax.axis_index("c")
        base = wid * ROWS_W
        idx_b = (idx0, idx1)
        rows_b = (rows0, rows1)
        gsem = (g0, g1)
        osem = (o0, o1)

        def gstart(s, b):
            row0 = base + s * R
            pltpu.sync_copy(x_hbm.at[pl.ds(row0, R), :], idx_b[b])
            for r in range(R):
                pltpu.async_copy(
                    table_hbm.at[idx_b[b].at[r]], rows_b[b].at[r], gsem[b])

        def gwait(b):
            for r in range(R):
                pltpu.make_async_copy(
                    table_hbm.at[idx_b[b].at[r]], rows_b[b].at[r],
                    gsem[b]).wait()

        def ostart(s, b):
            row0 = base + s * R
            pltpu.async_copy(
                rows_b[b], out_hbm.at[pl.ds(row0, R), :, :], osem[b])

        def owait(s, b):
            row0 = base + s * R
            pltpu.make_async_copy(
                rows_b[b], out_hbm.at[pl.ds(row0, R), :, :], osem[b]).wait()

        gstart(0, 0)

        def body(j, carry):
            for b in range(2):
                s = 2 * j + b
                nb = 1 - b
                pl.when(s >= 1)(lambda: owait(s - 1, nb))
                pl.when(s + 1 < NSTEP)(lambda: gstart(s + 1, nb))
                gwait(b)
                ostart(s, b)
            return carry

        lax.fori_loop(0, NSTEP // 2, body, 0)
        owait(NSTEP - 1, 1)

    return emb_kernel


_EMB = _make_kernel()


def kernel(x, weight):
    return _EMB(x, weight)
